# baseline (speedup 1.0000x reference)
import functools

import jax
import jax.numpy as jnp
from jax import lax
from jax.experimental import pallas as pl
from jax.experimental.pallas import tpu as pltpu
from jax.experimental.pallas import tpu_sc as plsc

N_NODES = 10000
N_EDGES = 320000
D_IN = 128
D_LAT = 128
D_EDGE = 16
D_OUT = 4
N_LAYERS = 3

NC = 2
NS = 16
NW = NC * NS
EPT = N_EDGES // NW
CH = 40
NCHUNK = EPT // CH
N_PAD = 10240
RPT = N_PAD // NS
ZROWS = 64
CNT_W = 128

_F32 = jnp.float32


def _elu(v):
    return jnp.where(v > 0, v, jnp.exp(v) - 1.0)


def _mlp_call(x, layers, acts, blk):
    n, din = x.shape
    dout = layers[-1][0].shape[1]

    def body(*refs):
        x_ref, o_ref = refs[0], refs[-1]
        h = x_ref[...]
        k = 1
        for i in range(len(layers)):
            w = refs[k][...]
            b = refs[k + 1][...]
            k += 2
            h = jnp.dot(h, w, preferred_element_type=_F32) + b
            if acts[i]:
                h = _elu(h)
        o_ref[...] = h

    in_specs = [pl.BlockSpec((blk, din), lambda i: (i, 0))]
    ops = [x]
    for (w, b) in layers:
        in_specs.append(pl.BlockSpec(w.shape, lambda i: (0, 0)))
        in_specs.append(pl.BlockSpec(b.shape, lambda i: (0, 0)))
        ops += [w, b]
    return pl.pallas_call(
        body,
        grid=(n // blk,),
        in_specs=in_specs,
        out_specs=pl.BlockSpec((blk, dout), lambda i: (i, 0)),
        out_shape=jax.ShapeDtypeStruct((n, dout), _F32),
    )(*ops)


def _proj_call(h, wi, wj):
    blk = 1000

    def body(h_ref, wi_ref, wj_ref, pi_ref, pj_ref):
        hh = h_ref[...]
        pi_ref[...] = jnp.dot(hh, wi_ref[...], preferred_element_type=_F32)
        pj_ref[...] = jnp.dot(hh, wj_ref[...], preferred_element_type=_F32)

    out_sds = jax.ShapeDtypeStruct((N_NODES, D_LAT), _F32)
    return pl.pallas_call(
        body,
        grid=(N_NODES // blk,),
        in_specs=[
            pl.BlockSpec((blk, D_LAT), lambda i: (i, 0)),
            pl.BlockSpec((D_LAT, D_LAT), lambda i: (0, 0)),
            pl.BlockSpec((D_LAT, D_LAT), lambda i: (0, 0)),
        ],
        out_specs=(
            pl.BlockSpec((blk, D_LAT), lambda i: (i, 0)),
            pl.BlockSpec((blk, D_LAT), lambda i: (i, 0)),
        ),
        out_shape=(out_sds, out_sds),
    )(h, wi, wj)


def _ae_call(edge_attr, we_all, b_list):
    blk = 2000
    nl = len(b_list)

    def body(*refs):
        ea = refs[0][...]
        w = refs[1][...]
        for l in range(nl):
            b = refs[2 + l][...]
            o = refs[2 + nl + l]
            o[...] = (
                jnp.dot(ea, w[:, l * D_LAT:(l + 1) * D_LAT],
                        preferred_element_type=_F32) + b
            )

    in_specs = [
        pl.BlockSpec((blk, D_EDGE), lambda i: (i, 0)),
        pl.BlockSpec(we_all.shape, lambda i: (0, 0)),
    ] + [pl.BlockSpec((1, D_LAT), lambda i: (0, 0))] * nl
    out_sds = jax.ShapeDtypeStruct((N_EDGES, D_LAT), _F32)
    return pl.pallas_call(
        body,
        grid=(N_EDGES // blk,),
        in_specs=in_specs,
        out_specs=(pl.BlockSpec((blk, D_LAT), lambda i: (i, 0)),) * nl,
        out_shape=(out_sds,) * nl,
    )(edge_attr, we_all, *b_list)


def _post_call(h, parts, cnts, w2, b2, g1h, g1a, g1b, g2, g2b, gscale, gbias):
    blk = 1000

    def body(h_ref, p_ref, c_ref, w2r, b2r, g1hr, g1ar, g1br, g2r, g2br,
             scr, bir, o_ref):
        s = p_ref[0] + p_ref[1]
        cnt = c_ref[0][:, 0:1] + c_ref[1][:, 0:1]
        denom = jnp.maximum(cnt, 1.0)
        mask = (cnt > 0).astype(_F32)
        aggr = jnp.dot(s, w2r[...], preferred_element_type=_F32) / denom \
            + b2r[...] * mask
        hh = h_ref[...]
        u = _elu(jnp.dot(hh, g1hr[...], preferred_element_type=_F32)
                 + jnp.dot(aggr, g1ar[...], preferred_element_type=_F32)
                 + g1br[...])
        hn = jnp.dot(u, g2r[...], preferred_element_type=_F32) + g2br[...]
        lane = lax.broadcasted_iota(jnp.int32, (blk, D_LAT), 1)
        grp0 = lane < (D_LAT // 2)
        half = float(D_LAT // 2)
        s_all = jnp.sum(hn, axis=1, keepdims=True)
        s0 = jnp.sum(jnp.where(grp0, hn, 0.0), axis=1, keepdims=True)
        mean = jnp.where(grp0, s0, s_all - s0) / half
        hn2 = hn * hn
        q_all = jnp.sum(hn2, axis=1, keepdims=True)
        q0 = jnp.sum(jnp.where(grp0, hn2, 0.0), axis=1, keepdims=True)
        msq = jnp.where(grp0, q0, q_all - q0) / half
        var = msq - mean * mean
        o_ref[...] = ((hn - mean) * lax.rsqrt(var + 1e-5)) * scr[...] + bir[...]

    wspec = pl.BlockSpec((D_LAT, D_LAT), lambda i: (0, 0))
    bspec = pl.BlockSpec((1, D_LAT), lambda i: (0, 0))
    return pl.pallas_call(
        body,
        grid=(N_NODES // blk,),
        in_specs=[
            pl.BlockSpec((blk, D_LAT), lambda i: (i, 0)),
            pl.BlockSpec((NC, blk, D_LAT), lambda i: (0, i, 0)),
            pl.BlockSpec((NC, blk, CNT_W), lambda i: (0, i, 0)),
            wspec, bspec, wspec, wspec, bspec, wspec, bspec, bspec, bspec,
        ],
        out_specs=pl.BlockSpec((blk, D_LAT), lambda i: (i, 0)),
        out_shape=jax.ShapeDtypeStruct((N_NODES, D_LAT), _F32),
    )(h, parts, cnts, w2, b2, g1h, g1a, g1b, g2, g2b, gscale, gbias)


_MESH = plsc.VectorSubcoreMesh(core_axis_name="c", subcore_axis_name="s")


@functools.partial(
    pl.kernel,
    out_type=jax.ShapeDtypeStruct((NC, N_PAD, CNT_W), _F32),
    mesh=_MESH,
    scratch_types=[
        pltpu.VMEM((CH,), jnp.int32),
        pltpu.VMEM((CH, CNT_W), _F32),
        pltpu.VMEM((ZROWS, CNT_W), _F32),
        pltpu.VMEM_SHARED((N_PAD, CNT_W), _F32),
    ],
)
def _counts_kernel(dst_hbm, out_hbm, dstb, onesb, zb, acc):
    cid = lax.axis_index("c")
    sid = lax.axis_index("s")
    wid = sid * NC + cid

    def fill(r, _):
        for c in range(CNT_W // 16):
            zb[r, pl.ds(c * 16, 16)] = jnp.zeros((16,), _F32)
        return 0

    lax.fori_loop(0, ZROWS, fill, 0)

    def fill1(r, _):
        for c in range(CNT_W // 16):
            onesb[r, pl.ds(c * 16, 16)] = jnp.ones((16,), _F32)
        return 0

    lax.fori_loop(0, CH, fill1, 0)

    for k in range(RPT // ZROWS):
        pltpu.sync_copy(zb, acc.at[pl.ds(sid * RPT + k * ZROWS, ZROWS)])
    plsc.subcore_barrier()

    ebase = wid * EPT

    def chunk(k, _):
        pltpu.sync_copy(dst_hbm.at[pl.ds(ebase + k * CH, CH)], dstb)
        pltpu.sync_copy(onesb, acc.at[dstb], add=True)
        return 0

    lax.fori_loop(0, NCHUNK, chunk, 0)
    plsc.subcore_barrier()

    for k in range(RPT // ZROWS):
        rows = pl.ds(sid * RPT + k * ZROWS, ZROWS)
        pltpu.sync_copy(acc.at[rows], out_hbm.at[cid, rows])


@functools.partial(
    pl.kernel,
    out_type=jax.ShapeDtypeStruct((NC, N_PAD, D_LAT), _F32),
    mesh=_MESH,
    scratch_types=[
        pltpu.VMEM((CH,), jnp.int32),
        pltpu.VMEM((CH,), jnp.int32),
        pltpu.VMEM((CH,), jnp.int32),
        pltpu.VMEM((CH,), jnp.int32),
        pltpu.VMEM((CH, D_LAT), _F32),
        pltpu.VMEM((CH, D_LAT), _F32),
        pltpu.VMEM((CH, D_LAT), _F32),
        pltpu.VMEM((CH, D_LAT), _F32),
        pltpu.VMEM((CH, D_LAT), _F32),
        pltpu.VMEM((CH, D_LAT), _F32),
        pltpu.VMEM((ZROWS, D_LAT), _F32),
        pltpu.VMEM_SHARED((N_PAD, D_LAT), _F32),
        pltpu.SemaphoreType.DMA,
        pltpu.SemaphoreType.DMA,
        pltpu.SemaphoreType.DMA,
        pltpu.SemaphoreType.DMA,
        pltpu.SemaphoreType.DMA,
        pltpu.SemaphoreType.DMA,
        pltpu.SemaphoreType.DMA,
        pltpu.SemaphoreType.DMA,
        pltpu.SemaphoreType.DMA,
        pltpu.SemaphoreType.DMA,
    ],
)
def _edge_kernel(dst_hbm, src_hbm, pi_hbm, pj_hbm, ae_hbm, out_hbm,
                 dstA, srcA, dstB, srcB, pibA, pjbA, aebA, pibB, pjbB, aebB,
                 zb, acc, idA1, idA2, idB1, idB2, sA1, sA2, sA3,
                 sB1, sB2, sB3):
    cid = lax.axis_index("c")
    sid = lax.axis_index("s")
    wid = sid * NC + cid

    def fill(r, _):
        for c in range(D_LAT // 16):
            zb[r, pl.ds(c * 16, 16)] = jnp.zeros((16,), _F32)
        return 0

    lax.fori_loop(0, ZROWS, fill, 0)
    for k in range(RPT // ZROWS):
        pltpu.sync_copy(zb, acc.at[pl.ds(sid * RPT + k * ZROWS, ZROWS)])
    plsc.subcore_barrier()

    ebase = wid * EPT
    KC = NCHUNK - 1

    def issue_idx(k, dstb, srcb, i1, i2):
        base = ebase + k * CH
        pltpu.async_copy(dst_hbm.at[pl.ds(base, CH)], dstb, i1)
        pltpu.async_copy(src_hbm.at[pl.ds(base, CH)], srcb, i2)

    def wait_idx(k, dstb, srcb, i1, i2):
        base = ebase + k * CH
        pltpu.make_async_copy(dst_hbm.at[pl.ds(base, CH)], dstb, i1).wait()
        pltpu.make_async_copy(src_hbm.at[pl.ds(base, CH)], srcb, i2).wait()

    def issue_gather(k, dstb, srcb, pib, pjb, aeb, s1, s2, s3):
        pltpu.async_copy(pi_hbm.at[dstb], pib, s1)
        pltpu.async_copy(pj_hbm.at[srcb], pjb, s2)
        pltpu.async_copy(ae_hbm.at[pl.ds(ebase + k * CH, CH)], aeb, s3)

    def wait_gather(k, dstb, srcb, pib, pjb, aeb, s1, s2, s3):
        pltpu.make_async_copy(pi_hbm.at[dstb], pib, s1).wait()
        pltpu.make_async_copy(pj_hbm.at[srcb], pjb, s2).wait()
        pltpu.make_async_copy(
            ae_hbm.at[pl.ds(ebase + k * CH, CH)], aeb, s3).wait()

    def comp_scatter(dstb, pib, pjb, aeb):
        def comp(r, _):
            for c in range(D_LAT // 16):
                sl = pl.ds(c * 16, 16)
                t = pib[r, sl] + pjb[r, sl] + aeb[r, sl]
                pib[r, sl] = jnp.where(t > 0, t, jnp.exp(t) - 1.0)
            return 0

        lax.fori_loop(0, CH, comp, 0)
        pltpu.sync_copy(pib, acc.at[dstb], add=True)

    A = (dstA, srcA, pibA, pjbA, aebA, sA1, sA2, sA3)
    B = (dstB, srcB, pibB, pjbB, aebB, sB1, sB2, sB3)

    def g_issue(k, S):
        issue_gather(k, S[0], S[1], S[2], S[3], S[4], S[5], S[6], S[7])

    def g_wait(k, S):
        wait_gather(k, S[0], S[1], S[2], S[3], S[4], S[5], S[6], S[7])

    pltpu.sync_copy(dst_hbm.at[pl.ds(ebase, CH)], dstA)
    pltpu.sync_copy(src_hbm.at[pl.ds(ebase, CH)], srcA)
    g_issue(0, A)
    issue_idx(1, dstB, srcB, idB1, idB2)

    def pair(p, _):
        e = 2 * p
        wait_idx(e + 1, dstB, srcB, idB1, idB2)
        g_issue(e + 1, B)
        g_wait(e, A)
        comp_scatter(dstA, pibA, pjbA, aebA)
        issue_idx(jnp.minimum(e + 2, KC), dstA, srcA, idA1, idA2)
        g_wait(e + 1, B)
        comp_scatter(dstB, pibB, pjbB, aebB)
        issue_idx(jnp.minimum(e + 3, KC), dstB, srcB, idB1, idB2)
        wait_idx(jnp.minimum(e + 2, KC), dstA, srcA, idA1, idA2)
        g_issue(jnp.minimum(e + 2, KC), A)
        return 0

    lax.fori_loop(0, NCHUNK // 2, pair, 0)
    g_wait(KC, A)
    wait_idx(KC, dstB, srcB, idB1, idB2)
    plsc.subcore_barrier()

    for k in range(RPT // ZROWS):
        rows = pl.ds(sid * RPT + k * ZROWS, ZROWS)
        pltpu.sync_copy(acc.at[rows], out_hbm.at[cid, rows])


def kernel(x, edge_index, edge_attr, params):
    src = edge_index[0]
    dst = edge_index[1]

    enc = [(w, b.reshape(1, -1)) for (w, b) in params["encoder"]]
    dec = [(w, b.reshape(1, -1)) for (w, b) in params["decoder"]]

    h = _mlp_call(x, enc, acts=[True, True, False], blk=1000)

    we_all = jnp.concatenate(
        [params["phi"][l][0][0][2 * D_LAT:] for l in range(N_LAYERS)], axis=1)
    b1_list = [params["phi"][l][0][1].reshape(1, -1) for l in range(N_LAYERS)]
    ae = _ae_call(edge_attr, we_all, b1_list)

    cnts = _counts_kernel(dst)

    gscale = params["gn_scale"].reshape(1, -1)
    gbias = params["gn_bias"].reshape(1, -1)

    for l in range(N_LAYERS):
        (w1, _b1), (w2, b2) = params["phi"][l]
        (g1, g1b), (g2, g2b) = params["gamma"][l]
        pi, pj = _proj_call(h, w1[:D_LAT], w1[D_LAT:2 * D_LAT])
        parts = _edge_kernel(dst, src, pi, pj, ae[l])
        h = _post_call(
            h, parts, cnts,
            w2, b2.reshape(1, -1),
            g1[:D_LAT], g1[D_LAT:], g1b.reshape(1, -1),
            g2, g2b.reshape(1, -1),
            gscale, gbias,
        )

    return _mlp_call(h, dec, acts=[True, True, False], blk=1000)
